# Initial kernel scaffold; baseline (speedup 1.0000x reference)
import jax
import jax.numpy as jnp
from jax.experimental import pallas as pl


def kernel(x, A):
    raise NotImplementedError("write your pallas kernel here")


# speedup vs baseline: 13.0933x; 13.0933x over previous
import functools

import jax
import jax.numpy as jnp
from jax.experimental import pallas as pl
from jax.experimental.pallas import tpu as pltpu

_N = 4096
_D = 128
_K = 10
_R = 256


def _body(x_ref, xt_ref, a_ref, out_ref, xw_ref, sqt_ref):
    i = pl.program_id(0)

    @pl.when(i == 0)
    def _init():
        xt = xt_ref[...]
        sqt_ref[...] = jnp.sum(xt * xt, axis=0, keepdims=True)
        xw_ref[...] = jnp.dot(x_ref[...], a_ref[...],
                              preferred_element_type=jnp.float32)

    x_blk = x_ref[pl.ds(i * _R, _R), :]
    sq_blk = jnp.sum(x_blk * x_blk, axis=1, keepdims=True)
    g = jnp.dot(x_blk, xt_ref[...], preferred_element_type=jnp.float32)
    s = sq_blk + sqt_ref[...] - 2.0 * g

    row = i * _R + jax.lax.broadcasted_iota(jnp.int32, (_R, _N), 0)
    col = jax.lax.broadcasted_iota(jnp.int32, (_R, _N), 1)
    inf = jnp.float32(jnp.inf)
    s = jnp.where(row == col, inf, s)

    m_mask = jnp.zeros((_R, _N), jnp.float32)
    for _ in range(_K):
        m = jnp.min(s, axis=1, keepdims=True)
        eq = s == m
        fi = jnp.min(jnp.where(eq, col, _N), axis=1, keepdims=True)
        hit = col == fi
        m_mask = jnp.where(hit, 1.0, m_mask)
        s = jnp.where(hit, inf, s)

    out_ref[...] = jnp.dot(m_mask, xw_ref[...],
                           preferred_element_type=jnp.float32)


@jax.jit
def kernel(x, A):
    xt = x.T
    grid = (_N // _R,)
    return pl.pallas_call(
        _body,
        grid=grid,
        in_specs=[
            pl.BlockSpec((_N, _D), lambda i: (0, 0)),
            pl.BlockSpec((_D, _N), lambda i: (0, 0)),
            pl.BlockSpec((_D, _D), lambda i: (0, 0)),
        ],
        out_specs=pl.BlockSpec((_R, _D), lambda i: (i, 0)),
        out_shape=jax.ShapeDtypeStruct((_N, _D), jnp.float32),
        scratch_shapes=[
            pltpu.VMEM((_N, _D), jnp.float32),
            pltpu.VMEM((1, _N), jnp.float32),
        ],
    )(x, xt, A)


# speedup vs baseline: 17.8198x; 1.3610x over previous
import functools

import jax
import jax.numpy as jnp
from jax.experimental import pallas as pl
from jax.experimental.pallas import tpu as pltpu

_N = 4096
_D = 128
_K = 10
_R = 256


def _body(x_ref, xt_ref, a_ref, out_ref, xw_ref, sqt_ref):
    i = pl.program_id(0)

    @pl.when(i == 0)
    def _init():
        xt = xt_ref[...]
        sqt_ref[...] = jnp.sum(xt * xt, axis=0, keepdims=True)
        xw_ref[...] = jnp.dot(x_ref[...], a_ref[...],
                              preferred_element_type=jnp.float32)

    x_blk = x_ref[pl.ds(i * _R, _R), :]
    sq_blk = jnp.sum(x_blk * x_blk, axis=1, keepdims=True)
    g = jnp.dot(x_blk, xt_ref[...], preferred_element_type=jnp.float32)
    s = sq_blk + sqt_ref[...] - 2.0 * g

    row = i * _R + jax.lax.broadcasted_iota(jnp.int32, (_R, _N), 0)
    col = jax.lax.broadcasted_iota(jnp.int32, (_R, _N), 1)
    inf = jnp.float32(jnp.inf)
    s0 = jnp.where(row == col, inf, s)

    s = s0
    kf = jnp.float32(_K)
    cum = jnp.zeros((_R, 1), jnp.float32)
    v_thr = jnp.zeros((_R, 1), jnp.float32)
    n_less = jnp.zeros((_R, 1), jnp.float32)
    c_thr = jnp.zeros((_R, 1), jnp.float32)
    for _ in range(_K):
        m = jnp.min(s, axis=1, keepdims=True)
        eq = s == m
        c = jnp.sum(jnp.where(eq, 1.0, 0.0), axis=1, keepdims=True)
        s = jnp.where(eq, inf, s)
        prev = cum
        cum = cum + c
        sel = (prev < kf) & (cum >= kf)
        v_thr = jnp.where(sel, m, v_thr)
        n_less = jnp.where(sel, prev, n_less)
        c_thr = jnp.where(sel, c, c_thr)

    frac = (kf - n_less) / c_thr
    m_mask = jnp.where(s0 < v_thr, 1.0, jnp.where(s0 == v_thr, frac, 0.0))

    out_ref[...] = jnp.dot(m_mask, xw_ref[...],
                           preferred_element_type=jnp.float32)


@jax.jit
def kernel(x, A):
    xt = x.T
    grid = (_N // _R,)
    return pl.pallas_call(
        _body,
        grid=grid,
        in_specs=[
            pl.BlockSpec((_N, _D), lambda i: (0, 0)),
            pl.BlockSpec((_D, _N), lambda i: (0, 0)),
            pl.BlockSpec((_D, _D), lambda i: (0, 0)),
        ],
        out_specs=pl.BlockSpec((_R, _D), lambda i: (i, 0)),
        out_shape=jax.ShapeDtypeStruct((_N, _D), jnp.float32),
        scratch_shapes=[
            pltpu.VMEM((_N, _D), jnp.float32),
            pltpu.VMEM((1, _N), jnp.float32),
        ],
    )(x, xt, A)


# speedup vs baseline: 23.7251x; 1.3314x over previous
import functools

import jax
import jax.numpy as jnp
from jax.experimental import pallas as pl
from jax.experimental.pallas import tpu as pltpu

_N = 4096
_D = 128
_K = 10
_R = 256


def _body(x_ref, xt_ref, a_ref, out_ref, xw_ref, sqt_ref):
    i = pl.program_id(0)

    @pl.when(i == 0)
    def _init():
        xt = xt_ref[...]
        sqt_ref[...] = jnp.sum(xt * xt, axis=0, keepdims=True)
        xw_ref[...] = jnp.dot(x_ref[...], a_ref[...],
                              preferred_element_type=jnp.float32)

    x_blk = x_ref[pl.ds(i * _R, _R), :]
    sq_blk = jnp.sum(x_blk * x_blk, axis=1, keepdims=True)
    g = jnp.dot(x_blk, xt_ref[...], preferred_element_type=jnp.float32)
    s = sq_blk + sqt_ref[...] - 2.0 * g

    row = i * _R + jax.lax.broadcasted_iota(jnp.int32, (_R, _N), 0)
    col = jax.lax.broadcasted_iota(jnp.int32, (_R, _N), 1)
    inf = jnp.float32(jnp.inf)
    s0 = jnp.where(row == col, inf, s)

    q = _N // 4
    a, b = s0[:, :q], s0[:, q:2 * q]
    c_, d = s0[:, 2 * q:3 * q], s0[:, 3 * q:]
    a, b = jnp.minimum(a, b), jnp.maximum(a, b)
    c_, d = jnp.minimum(c_, d), jnp.maximum(c_, d)
    a, c_ = jnp.minimum(a, c_), jnp.maximum(a, c_)
    b, d = jnp.minimum(b, d), jnp.maximum(b, d)
    b, c_ = jnp.minimum(b, c_), jnp.maximum(b, c_)

    kf = jnp.float32(_K)
    cum = jnp.zeros((_R, 1), jnp.float32)
    v_thr = jnp.zeros((_R, 1), jnp.float32)
    for _ in range(_K):
        m = jnp.min(a, axis=1, keepdims=True)
        eq = a == m
        cnt = jnp.sum(jnp.where(eq, 1.0, 0.0), axis=1, keepdims=True)
        a = jnp.where(eq, b, a)
        b = jnp.where(eq, c_, b)
        c_ = jnp.where(eq, d, c_)
        d = jnp.where(eq, inf, d)
        prev = cum
        cum = cum + cnt
        sel = (prev < kf) & (cum >= kf)
        v_thr = jnp.where(sel, m, v_thr)

    lt = s0 < v_thr
    eqt = s0 == v_thr
    cl = jnp.sum(jnp.where(lt, 1.0, 0.0), axis=1, keepdims=True)
    ce = jnp.sum(jnp.where(eqt, 1.0, 0.0), axis=1, keepdims=True)
    frac = (kf - cl) / ce
    m_mask = jnp.where(lt, 1.0, jnp.where(eqt, frac, 0.0))

    out_ref[...] = jnp.dot(m_mask, xw_ref[...],
                           preferred_element_type=jnp.float32)


@jax.jit
def kernel(x, A):
    xt = x.T
    grid = (_N // _R,)
    return pl.pallas_call(
        _body,
        grid=grid,
        in_specs=[
            pl.BlockSpec((_N, _D), lambda i: (0, 0)),
            pl.BlockSpec((_D, _N), lambda i: (0, 0)),
            pl.BlockSpec((_D, _D), lambda i: (0, 0)),
        ],
        out_specs=pl.BlockSpec((_R, _D), lambda i: (i, 0)),
        out_shape=jax.ShapeDtypeStruct((_N, _D), jnp.float32),
        scratch_shapes=[
            pltpu.VMEM((_N, _D), jnp.float32),
            pltpu.VMEM((1, _N), jnp.float32),
        ],
    )(x, xt, A)


# speedup vs baseline: 27.1017x; 1.1423x over previous
import functools

import jax
import jax.numpy as jnp
from jax.experimental import pallas as pl
from jax.experimental.pallas import tpu as pltpu

_N = 4096
_D = 128
_K = 10
_R = 256


def _body(x_ref, xt_ref, a_ref, out_ref, xw_ref, sqt_ref):
    i = pl.program_id(0)

    @pl.when(i == 0)
    def _init():
        xt = xt_ref[...]
        sqt_ref[...] = jnp.sum(xt * xt, axis=0, keepdims=True)
        xw_ref[...] = jnp.dot(x_ref[...], a_ref[...],
                              preferred_element_type=jnp.float32)

    x_blk = x_ref[pl.ds(i * _R, _R), :]
    sq_blk = jnp.sum(x_blk * x_blk, axis=1, keepdims=True)
    g = jnp.dot(x_blk, xt_ref[...], preferred_element_type=jnp.float32)
    s = sq_blk + sqt_ref[...] - 2.0 * g

    row = i * _R + jax.lax.broadcasted_iota(jnp.int32, (_R, _N), 0)
    col = jax.lax.broadcasted_iota(jnp.int32, (_R, _N), 1)
    inf = jnp.float32(jnp.inf)
    s0 = jnp.where(row == col, inf, s)

    q = _N // 8
    p = [s0[:, j * q:(j + 1) * q] for j in range(8)]

    def cex(i, j):
        p[i], p[j] = jnp.minimum(p[i], p[j]), jnp.maximum(p[i], p[j])

    for i, j in ((0, 1), (2, 3), (0, 2), (1, 3), (1, 2),
                 (4, 5), (6, 7), (4, 6), (5, 7), (5, 6),
                 (0, 4), (1, 5), (2, 6), (3, 7),
                 (2, 4), (3, 5),
                 (1, 2), (3, 4), (5, 6)):
        cex(i, j)

    kf = jnp.float32(_K)
    neginf = jnp.float32(-jnp.inf)
    cum = jnp.zeros((_R, 1), jnp.float32)
    v_thr = jnp.zeros((_R, 1), jnp.float32)
    cl_bk = jnp.zeros((_R, 1), jnp.float32)
    run_base = jnp.zeros((_R, 1), jnp.float32)
    m_prev = jnp.full((_R, 1), neginf)
    for _ in range(_K):
        m = jnp.min(p[0], axis=1, keepdims=True)
        eq = p[0] == m
        cnt = jnp.sum(jnp.where(eq, 1.0, 0.0), axis=1, keepdims=True)
        for j in range(7):
            p[j] = jnp.where(eq, p[j + 1], p[j])
        p[7] = jnp.where(eq, inf, p[7])
        run_base = jnp.where(m > m_prev, cum, run_base)
        newcum = cum + cnt
        sel = (cum < kf) & (newcum >= kf)
        v_thr = jnp.where(sel, m, v_thr)
        cl_bk = jnp.where(sel, run_base, cl_bk)
        cum = newcum
        m_prev = m

    lt = s0 < v_thr
    eqt = s0 == v_thr
    ce = jnp.sum(jnp.where(eqt, 1.0, 0.0), axis=1, keepdims=True)
    frac = (kf - cl_bk) / ce
    m_mask = jnp.where(lt, 1.0, jnp.where(eqt, frac, 0.0))

    out_ref[...] = jnp.dot(m_mask, xw_ref[...],
                           preferred_element_type=jnp.float32)


@jax.jit
def kernel(x, A):
    xt = x.T
    grid = (_N // _R,)
    return pl.pallas_call(
        _body,
        grid=grid,
        in_specs=[
            pl.BlockSpec((_N, _D), lambda i: (0, 0)),
            pl.BlockSpec((_D, _N), lambda i: (0, 0)),
            pl.BlockSpec((_D, _D), lambda i: (0, 0)),
        ],
        out_specs=pl.BlockSpec((_R, _D), lambda i: (i, 0)),
        out_shape=jax.ShapeDtypeStruct((_N, _D), jnp.float32),
        scratch_shapes=[
            pltpu.VMEM((_N, _D), jnp.float32),
            pltpu.VMEM((1, _N), jnp.float32),
        ],
    )(x, xt, A)


# speedup vs baseline: 27.4609x; 1.0133x over previous
import functools

import jax
import jax.numpy as jnp
from jax.experimental import pallas as pl
from jax.experimental.pallas import tpu as pltpu

_N = 4096
_D = 128
_K = 10
_R = 512


def _body(x_ref, xt_ref, a_ref, out_ref, xw_ref, sqt_ref):
    i = pl.program_id(0)

    @pl.when(i == 0)
    def _init():
        xt = xt_ref[...]
        sqt_ref[...] = jnp.sum(xt * xt, axis=0, keepdims=True)
        xw_ref[...] = jnp.dot(x_ref[...], a_ref[...],
                              preferred_element_type=jnp.float32)

    x_blk = x_ref[pl.ds(i * _R, _R), :]
    sq_blk = jnp.sum(x_blk * x_blk, axis=1, keepdims=True)
    g = jnp.dot(x_blk, xt_ref[...], preferred_element_type=jnp.float32)
    s = sq_blk + sqt_ref[...] - 2.0 * g

    row = i * _R + jax.lax.broadcasted_iota(jnp.int32, (_R, _N), 0)
    col = jax.lax.broadcasted_iota(jnp.int32, (_R, _N), 1)
    inf = jnp.float32(jnp.inf)
    s0 = jnp.where(row == col, inf, s)

    q = _N // 8
    p = [s0[:, j * q:(j + 1) * q] for j in range(8)]

    def cex(i, j):
        p[i], p[j] = jnp.minimum(p[i], p[j]), jnp.maximum(p[i], p[j])

    for i, j in ((0, 1), (2, 3), (0, 2), (1, 3), (1, 2),
                 (4, 5), (6, 7), (4, 6), (5, 7), (5, 6),
                 (0, 4), (1, 5), (2, 6), (3, 7),
                 (2, 4), (3, 5),
                 (1, 2), (3, 4), (5, 6)):
        cex(i, j)

    kf = jnp.float32(_K)
    neginf = jnp.float32(-jnp.inf)
    cum = jnp.zeros((_R, 1), jnp.float32)
    v_thr = jnp.zeros((_R, 1), jnp.float32)
    cl_bk = jnp.zeros((_R, 1), jnp.float32)
    run_base = jnp.zeros((_R, 1), jnp.float32)
    m_prev = jnp.full((_R, 1), neginf)
    for _ in range(_K):
        m = jnp.min(p[0], axis=1, keepdims=True)
        eq = p[0] == m
        cnt = jnp.sum(jnp.where(eq, 1.0, 0.0), axis=1, keepdims=True)
        for j in range(7):
            p[j] = jnp.where(eq, p[j + 1], p[j])
        p[7] = jnp.where(eq, inf, p[7])
        run_base = jnp.where(m > m_prev, cum, run_base)
        newcum = cum + cnt
        sel = (cum < kf) & (newcum >= kf)
        v_thr = jnp.where(sel, m, v_thr)
        cl_bk = jnp.where(sel, run_base, cl_bk)
        cum = newcum
        m_prev = m

    lt = s0 < v_thr
    eqt = s0 == v_thr
    ce = jnp.sum(jnp.where(eqt, 1.0, 0.0), axis=1, keepdims=True)
    frac = (kf - cl_bk) / ce
    m_mask = jnp.where(lt, 1.0, jnp.where(eqt, frac, 0.0))

    out_ref[...] = jnp.dot(m_mask, xw_ref[...],
                           preferred_element_type=jnp.float32)


@jax.jit
def kernel(x, A):
    xt = x.T
    grid = (_N // _R,)
    return pl.pallas_call(
        _body,
        grid=grid,
        in_specs=[
            pl.BlockSpec((_N, _D), lambda i: (0, 0)),
            pl.BlockSpec((_D, _N), lambda i: (0, 0)),
            pl.BlockSpec((_D, _D), lambda i: (0, 0)),
        ],
        out_specs=pl.BlockSpec((_R, _D), lambda i: (i, 0)),
        out_shape=jax.ShapeDtypeStruct((_N, _D), jnp.float32),
        scratch_shapes=[
            pltpu.VMEM((_N, _D), jnp.float32),
            pltpu.VMEM((1, _N), jnp.float32),
        ],
    )(x, xt, A)
